# baseline (speedup 1.0000x reference)
import functools

import jax
import jax.numpy as jnp
from jax import lax
from jax.experimental import pallas as pl
from jax.experimental.pallas import tpu as pltpu
from jax.experimental.pallas import tpu_sc as plsc

N = 10000
E = 160000
HID = 128
H = 8
DH = 16
NL = 2
OUT = 64

NC = 2
NS = 16
NW = NC * NS
CH = 128
NCHUNK = E // CH
RPT = N // NS
ACCW = 144

BR = 2000
NBLK = N // BR

_F32 = jnp.float32
_NEG = -3.0e38


def _inproj_body(x_ref, w_ref, b_ref, o_ref):
    y = jnp.dot(x_ref[0], w_ref[0], preferred_element_type=_F32) + b_ref[0]
    o_ref[0] = jnp.maximum(y, 0.0)


def _input_proj(X, Win, b_in3):
    return pl.pallas_call(
        _inproj_body,
        grid=(2, NBLK),
        in_specs=[
            pl.BlockSpec((1, BR, HID), lambda t, r: (t, r, 0)),
            pl.BlockSpec((1, HID, HID), lambda t, r: (t, 0, 0)),
            pl.BlockSpec((1, 1, HID), lambda t, r: (t, 0, 0)),
        ],
        out_specs=pl.BlockSpec((1, BR, HID), lambda t, r: (t, r, 0)),
        out_shape=jax.ShapeDtypeStruct((2, N, HID), _F32),
    )(X, Win, b_in3)


def _effw_body(wq_ref, wk_ref, wv_ref, bda_ref, bdm_ref, mu_ref,
               oq_ref, ok_ref, ov_ref):
    r = lax.broadcasted_iota(jnp.int32, (H, HID), 0)
    c = lax.broadcasted_iota(jnp.int32, (H, HID), 1)
    rep = (c // DH == r).astype(_F32)
    scale = mu_ref[0, 0] * (1.0 / (DH ** 0.5))
    srep = jnp.dot(scale, rep, preferred_element_type=_F32)
    oq_ref[0, 0] = wq_ref[0, 0] * srep
    ok_ref[0, 0] = jnp.dot(wk_ref[0, 0], bda_ref[0, 0],
                           preferred_element_type=_F32)
    ov_ref[0, 0] = jnp.dot(wv_ref[0, 0], bdm_ref[0, 0],
                           preferred_element_type=_F32)


def _eff_weights(Wq, Wk, Wv, BDa, BDm, mu4):
    w_spec_q = pl.BlockSpec((1, 1, HID, HID), lambda l, e: (l, 1 - e, 0, 0))
    w_spec = pl.BlockSpec((1, 1, HID, HID), lambda l, e: (l, e, 0, 0))
    return pl.pallas_call(
        _effw_body,
        grid=(NL, 2),
        in_specs=[w_spec_q, w_spec, w_spec, w_spec, w_spec,
                  pl.BlockSpec((1, 1, 1, H), lambda l, e: (l, e, 0, 0))],
        out_specs=[w_spec, w_spec, w_spec],
        out_shape=[jax.ShapeDtypeStruct((NL, 2, HID, HID), _F32)] * 3,
    )(Wq, Wk, Wv, BDa, BDm, mu4)


def _proj_body(xq_ref, xkv_ref, wq_ref, wk_ref, wv_ref, q_ref, k_ref, v_ref):
    q_ref[0] = jnp.dot(xq_ref[0], wq_ref[0], preferred_element_type=_F32)
    k_ref[0] = jnp.dot(xkv_ref[0], wk_ref[0], preferred_element_type=_F32)
    v_ref[0] = jnp.dot(xkv_ref[0], wv_ref[0], preferred_element_type=_F32)


def _proj(xs, Wqe_l, Wke_l, Wve_l):
    x_spec_q = pl.BlockSpec((1, BR, HID), lambda e, r: (1 - e, r, 0))
    x_spec = pl.BlockSpec((1, BR, HID), lambda e, r: (e, r, 0))
    w_spec = pl.BlockSpec((1, HID, HID), lambda e, r: (e, 0, 0))
    o_spec = pl.BlockSpec((1, BR, HID), lambda e, r: (e, r, 0))
    return pl.pallas_call(
        _proj_body,
        grid=(2, NBLK),
        in_specs=[x_spec_q, x_spec, w_spec, w_spec, w_spec],
        out_specs=[o_spec, o_spec, o_spec],
        out_shape=[jax.ShapeDtypeStruct((2, N, HID), _F32)] * 3,
    )(xs, xs, Wqe_l, Wke_l, Wve_l)


def _upd_body(n0_ref, d0_ref, n1_ref, d1_ref, xs_ref, wa_ref, beta_ref, o_ref):
    r = lax.broadcasted_iota(jnp.int32, (DH, HID), 0)
    c = lax.broadcasted_iota(jnp.int32, (DH, HID), 1)
    rep = (c // DH == r).astype(_F32)
    for t in range(2):
        num = (n0_ref, n1_ref)[t][0]
        den = (d0_ref, d1_ref)[t][0]
        deni = 1.0 / (den + 1e-9)
        deni_rep = jnp.dot(deni, rep, preferred_element_type=_F32)
        agg = num * deni_rep
        g = jax.nn.gelu(agg)
        a = jnp.dot(g, wa_ref[t], preferred_element_type=_F32)
        b = beta_ref[t][None, :]
        o_ref[t] = b * a + (1.0 - b) * xs_ref[t]


def _update(Pn, Pd, xs, Wa_l, beta2d):
    n0_spec = pl.BlockSpec((1, BR, HID), lambda r: (1, r, 0))
    d0_spec = pl.BlockSpec((1, BR, DH), lambda r: (1, r, 0))
    n1_spec = pl.BlockSpec((1, BR, HID), lambda r: (0, r, 0))
    d1_spec = pl.BlockSpec((1, BR, DH), lambda r: (0, r, 0))
    return pl.pallas_call(
        _upd_body,
        grid=(NBLK,),
        in_specs=[n0_spec, d0_spec, n1_spec, d1_spec,
                  pl.BlockSpec((2, BR, HID), lambda r: (0, r, 0)),
                  pl.BlockSpec((2, HID, HID), lambda r: (0, 0, 0)),
                  pl.BlockSpec((2, HID), lambda r: (0, 0))],
        out_specs=pl.BlockSpec((2, BR, HID), lambda r: (0, r, 0)),
        out_shape=jax.ShapeDtypeStruct((2, N, HID), _F32),
    )(Pn, Pd, Pn, Pd, xs, Wa_l, beta2d)


def _final_body(x_ref, w_ref, b_ref, o_ref):
    o_ref[...] = jnp.dot(x_ref[0], w_ref[...],
                         preferred_element_type=_F32) + b_ref[...]


def _final(xs, Wout, b_out2):
    return pl.pallas_call(
        _final_body,
        grid=(NBLK,),
        in_specs=[pl.BlockSpec((1, BR, HID), lambda r: (0, r, 0)),
                  pl.BlockSpec((HID, OUT), lambda r: (0, 0)),
                  pl.BlockSpec((1, OUT), lambda r: (0, 0))],
        out_specs=pl.BlockSpec((BR, OUT), lambda r: (r, 0)),
        out_shape=jax.ShapeDtypeStruct((N, OUT), _F32),
    )(xs, Wout, b_out2)


_sc_mesh = plsc.VectorSubcoreMesh(core_axis_name="c", subcore_axis_name="s",
                                  num_cores=NC, num_subcores=NS)
_sc_params = pltpu.CompilerParams(needs_layout_passes=False,
                                  use_tc_tiling_on_sc=False)


EPT = E // NS
CHM = 40
EPP = 10080
NCHT = EPP // CHM
UNR = 6
NIT = NCHT // UNR
ISZ = UNR * CHM


@functools.partial(
    pl.kernel,
    out_type=(jax.ShapeDtypeStruct((NC * N, HID), _F32),
              jax.ShapeDtypeStruct((NC * N, DH), _F32)),
    mesh=_sc_mesh,
    scratch_types=[
        pltpu.VMEM((2, ISZ), jnp.int32),
        pltpu.VMEM((2, ISZ), jnp.int32),
        pltpu.VMEM((3, CHM), jnp.int32),
        pltpu.VMEM((2, CHM), jnp.int32),
        pltpu.VMEM((2, CHM), jnp.int32),
        pltpu.VMEM((3, CHM), jnp.int32),
        pltpu.VMEM((2, CHM, HID), _F32),
        pltpu.VMEM((2, CHM, HID), _F32),
        pltpu.VMEM((3, CHM, HID), _F32),
        pltpu.VMEM((3, CHM, DH), _F32),
        pltpu.VMEM((256,), _F32),
        pltpu.VMEM_SHARED((N + 8, HID), _F32),
        pltpu.VMEM_SHARED((N + 8, DH), _F32),
        pltpu.SemaphoreType.DMA,
        pltpu.SemaphoreType.DMA,
        pltpu.SemaphoreType.DMA,
        pltpu.SemaphoreType.DMA,
        pltpu.SemaphoreType.DMA,
        pltpu.SemaphoreType.DMA,
        pltpu.SemaphoreType.DMA,
        pltpu.SemaphoreType.DMA,
        pltpu.SemaphoreType.DMA,
        pltpu.SemaphoreType.DMA,
        pltpu.SemaphoreType.DMA,
        pltpu.SemaphoreType.DMA,
        pltpu.SemaphoreType.DMA,
        pltpu.SemaphoreType.DMA,
        pltpu.SemaphoreType.DMA,
    ],
    compiler_params=_sc_params,
)
def _sc_edge(qtab, ktab, vtab, src, dst, zeros_n, zeros_d, outn, outd,
             rawS, rawD, sx, gq, gk, gv, qrows, krows, vrows, exden, stage,
             accn, accd, sq0, sq1, sk0, sk1, sv0, sv1, sv2,
             sa0, sa1, sa2, sb0, sb1, sb2, si0, si1):
    cid = lax.axis_index("c")
    sid = lax.axis_index("s")
    semq = (sq0, sq1)
    semk = (sk0, sk1)
    semv = (sv0, sv1, sv2)
    sema = (sa0, sa1, sa2)
    semb = (sb0, sb1, sb2)
    semi = (si0, si1)
    toff = cid * (NS * EPP) + sid * EPP

    r0 = sid * RPT
    pltpu.sync_copy(zeros_n.at[pl.ds(r0, RPT)], accn.at[pl.ds(r0, RPT)])
    pltpu.sync_copy(zeros_d.at[pl.ds(r0, RPT)], accd.at[pl.ds(r0, RPT)])
    plsc.subcore_barrier()

    lanes = lax.iota(jnp.int32, 16)
    g15 = lanes * 16 + 15
    rowoff = cid * N
    offs = list(range(0, CHM - 16, 16)) + [CHM - 16]

    def issue_idx_load(it, ib):
        base = toff + it * ISZ
        pltpu.async_copy(src.at[pl.ds(base, ISZ)], rawS.at[ib], semi[ib])
        pltpu.async_copy(dst.at[pl.ds(base, ISZ)], rawD.at[ib], semi[ib])

    def wait_idx(ib):
        pltpu.make_async_copy(src.at[pl.ds(toff, ISZ)], rawS.at[ib],
                              semi[ib]).wait()
        pltpu.make_async_copy(dst.at[pl.ds(toff, ISZ)], rawD.at[ib],
                              semi[ib]).wait()

    def stage_qk(loc, ib, b):
        for o in offs:
            dv_ = rawD[ib, pl.ds(loc * CHM + o, 16)]
            sv_ = rawS[ib, pl.ds(loc * CHM + o, 16)]
            gq[b, pl.ds(o, 16)] = jnp.minimum(dv_, N - 1) + rowoff
            gk[b, pl.ds(o, 16)] = jnp.minimum(sv_, N - 1) + rowoff
        pltpu.async_copy(qtab.at[gq.at[b]], qrows.at[b], semq[b])
        pltpu.async_copy(ktab.at[gk.at[b]], krows.at[b], semk[b])

    def stage_v(loc, ib, t):
        for o in offs:
            dv_ = rawD[ib, pl.ds(loc * CHM + o, 16)]
            sv_ = rawS[ib, pl.ds(loc * CHM + o, 16)]
            sx[t, pl.ds(o, 16)] = dv_
            gv[t, pl.ds(o, 16)] = jnp.minimum(sv_, N - 1) + rowoff
        pltpu.async_copy(vtab.at[gv.at[t]], vrows.at[t], semv[t])

    def wait_gathers_qk(b):
        pltpu.make_async_copy(qtab.at[gq.at[b]], qrows.at[b], semq[b]).wait()
        pltpu.make_async_copy(ktab.at[gk.at[b]], krows.at[b], semk[b]).wait()

    def wait_gather_v(t):
        pltpu.make_async_copy(vtab.at[gv.at[t]], vrows.at[t], semv[t]).wait()

    def issue_scatter(t):
        pltpu.async_copy(vrows.at[t], accn.at[sx.at[t]], sema[t], add=True)
        pltpu.async_copy(exden.at[t], accd.at[sx.at[t]], semb[t], add=True)

    def wait_scatter(t):
        pltpu.make_async_copy(vrows.at[t], accn.at[sx.at[t]], sema[t]).wait()
        pltpu.make_async_copy(exden.at[t], accd.at[sx.at[t]], semb[t]).wait()

    def compute(b, t):
        qr, kr, vr, er = qrows.at[b], krows.at[b], vrows.at[t], exden.at[t]

        def edge_body(i, cc):
            for h in range(H):
                qv = qr[i, pl.ds(h * DH, DH)]
                kv = kr[i, pl.ds(h * DH, DH)]
                stage[pl.ds(h * DH, DH)] = plsc.cumsum(qv * kv)
            tot = plsc.load_gather(stage, [g15])
            sc = jnp.clip(tot, -60.0, 60.0)
            ex = jnp.where(lanes < H, jnp.exp(sc), 0.0)
            er[i, pl.ds(0, DH)] = ex
            for h in range(H):
                vr[i, pl.ds(h * DH, DH)] = vr[i, pl.ds(h * DH, DH)] * ex[h]
            return cc

        lax.fori_loop(0, CHM, edge_body, 0)

    issue_idx_load(0, 0)
    wait_idx(0)
    issue_idx_load(1, 1)
    stage_qk(0, 0, 0)
    stage_v(0, 0, 0)
    stage_qk(1, 0, 1)

    def iter_pair_body(itp, carry):
        for ip in range(2):
            it = 2 * itp + ip
            ib, nxt = ip, 1 - ip

            def _issue_next_idx():
                issue_idx_load(it + 1, nxt)
            if ip == 0:
                @pl.when(itp >= 1)
                def _():
                    _issue_next_idx()
            else:
                @pl.when(itp < NIT // 2 - 1)
                def _():
                    _issue_next_idx()

            for u in range(UNR):
                b = u % 2
                t = u % 3
                tp1 = (u + 1) % 3
                wait_gathers_qk(b)
                wait_gather_v(t)
                compute(b, t)
                if ip == 0 and u < 2:
                    @pl.when(itp >= 1)
                    def _():
                        wait_scatter(tp1)
                else:
                    wait_scatter(tp1)
                if u < UNR - 1:
                    stage_v(u + 1, ib, tp1)
                else:
                    def _stage_v_next(tp1_=tp1):
                        stage_v(0, nxt, tp1_)
                    if ip == 0:
                        _stage_v_next()
                    else:
                        @pl.when(itp < NIT // 2 - 1)
                        def _():
                            _stage_v_next()
                if u < UNR - 2:
                    stage_qk(u + 2, ib, b)
                elif u == UNR - 2:
                    def _stage_from_next(b_=b):
                        wait_idx(nxt)
                        stage_qk(0, nxt, b_)
                    if ip == 0:
                        _stage_from_next()
                    else:
                        @pl.when(itp < NIT // 2 - 1)
                        def _():
                            _stage_from_next()
                else:
                    def _stage_from_next2(b_=b):
                        stage_qk(1, nxt, b_)
                    if ip == 0:
                        _stage_from_next2()
                    else:
                        @pl.when(itp < NIT // 2 - 1)
                        def _():
                            _stage_from_next2()
                issue_scatter(t)
        return carry

    lax.fori_loop(0, NIT // 2, iter_pair_body, 0)
    wait_scatter((NCHT - 2) % 3)
    wait_scatter((NCHT - 1) % 3)

    plsc.subcore_barrier()
    pltpu.sync_copy(accn.at[pl.ds(r0, RPT)],
                    outn.at[pl.ds(cid * N + r0, RPT)])
    pltpu.sync_copy(accd.at[pl.ds(r0, RPT)],
                    outd.at[pl.ds(cid * N + r0, RPT)])


def kernel(x_user, x_item, edge_index_u2i, edge_index_i2u, Win, b_in,
           Wk, Wq, Wv, Wa, a_rel, m_rel, mu, skip, Wout, b_out):
    X = jnp.stack([x_user, x_item])
    xs = _input_proj(X, Win, b_in.reshape(2, 1, HID))

    eye = jnp.eye(H, dtype=_F32)
    def _bd(a):
        a6 = a[:, :, :, :, None, :] * eye[None, None, :, None, :, None]
        return a6.reshape(NL, 2, HID, HID)

    Wqe, Wke, Wve = _eff_weights(Wq, Wk, Wv, _bd(a_rel), _bd(m_rel),
                                 mu.reshape(NL, 2, 1, H))

    def _pad_edges(ei):
        sp = jnp.pad(ei[0].reshape(NS, EPT), ((0, 0), (0, EPP - EPT)))
        dp = jnp.pad(ei[1].reshape(NS, EPT), ((0, 0), (0, EPP - EPT)),
                     constant_values=N)
        return sp.reshape(-1), dp.reshape(-1)

    s0, d0 = _pad_edges(edge_index_u2i)
    s1, d1 = _pad_edges(edge_index_i2u)
    srccat = jnp.concatenate([s0, s1])
    dstcat = jnp.concatenate([d0, d1])
    zeros_n = jnp.zeros((N, HID), _F32)
    zeros_d = jnp.zeros((N, DH), _F32)
    beta = jax.nn.sigmoid(skip)

    for l in range(NL):
        Q, K, V = _proj(xs, Wqe[l], Wke[l], Wve[l])
        Pn, Pd = _sc_edge(Q.reshape(NC * N, HID), K.reshape(NC * N, HID),
                          V.reshape(NC * N, HID), srccat, dstcat,
                          zeros_n, zeros_d)
        beta2d = jnp.broadcast_to(beta[l][:, None], (2, HID))
        xs = _update(Pn.reshape(NC, N, HID), Pd.reshape(NC, N, DH),
                     xs, Wa[l], beta2d)

    return _final(xs, Wout, b_out.reshape(1, OUT))


# speedup vs baseline: 1.2064x; 1.2064x over previous
import functools

import jax
import jax.numpy as jnp
from jax import lax
from jax.experimental import pallas as pl
from jax.experimental.pallas import tpu as pltpu
from jax.experimental.pallas import tpu_sc as plsc

N = 10000
E = 160000
HID = 128
H = 8
DH = 16
NL = 2
OUT = 64

NC = 2
NS = 16
NW = NC * NS
CH = 128
NCHUNK = E // CH
RPT = N // NS
ACCW = 144

BR = 2000
NBLK = N // BR

_F32 = jnp.float32
_NEG = -3.0e38


def _inproj_body(x_ref, w_ref, b_ref, o_ref):
    y = jnp.dot(x_ref[0], w_ref[0], preferred_element_type=_F32) + b_ref[0]
    o_ref[0] = jnp.maximum(y, 0.0)


def _input_proj(X, Win, b_in3):
    return pl.pallas_call(
        _inproj_body,
        grid=(2, NBLK),
        in_specs=[
            pl.BlockSpec((1, BR, HID), lambda t, r: (t, r, 0)),
            pl.BlockSpec((1, HID, HID), lambda t, r: (t, 0, 0)),
            pl.BlockSpec((1, 1, HID), lambda t, r: (t, 0, 0)),
        ],
        out_specs=pl.BlockSpec((1, BR, HID), lambda t, r: (t, r, 0)),
        out_shape=jax.ShapeDtypeStruct((2, N, HID), _F32),
    )(X, Win, b_in3)


def _effw_body(wq_ref, wk_ref, wv_ref, bda_ref, bdm_ref, mu_ref,
               oq_ref, ok_ref, ov_ref):
    r = lax.broadcasted_iota(jnp.int32, (H, HID), 0)
    c = lax.broadcasted_iota(jnp.int32, (H, HID), 1)
    rep = (c // DH == r).astype(_F32)
    scale = mu_ref[0, 0] * (1.0 / (DH ** 0.5))
    srep = jnp.dot(scale, rep, preferred_element_type=_F32)
    oq_ref[0, 0] = wq_ref[0, 0] * srep
    ok_ref[0, 0] = jnp.dot(wk_ref[0, 0], bda_ref[0, 0],
                           preferred_element_type=_F32)
    ov_ref[0, 0] = jnp.dot(wv_ref[0, 0], bdm_ref[0, 0],
                           preferred_element_type=_F32)


def _eff_weights(Wq, Wk, Wv, BDa, BDm, mu4):
    w_spec_q = pl.BlockSpec((1, 1, HID, HID), lambda l, e: (l, 1 - e, 0, 0))
    w_spec = pl.BlockSpec((1, 1, HID, HID), lambda l, e: (l, e, 0, 0))
    return pl.pallas_call(
        _effw_body,
        grid=(NL, 2),
        in_specs=[w_spec_q, w_spec, w_spec, w_spec, w_spec,
                  pl.BlockSpec((1, 1, 1, H), lambda l, e: (l, e, 0, 0))],
        out_specs=[w_spec, w_spec, w_spec],
        out_shape=[jax.ShapeDtypeStruct((NL, 2, HID, HID), _F32)] * 3,
    )(Wq, Wk, Wv, BDa, BDm, mu4)


def _proj_body(xq_ref, xkv_ref, wq_ref, wk_ref, wv_ref, q_ref, k_ref, v_ref):
    q_ref[0] = jnp.dot(xq_ref[0], wq_ref[0], preferred_element_type=_F32)
    k_ref[0] = jnp.dot(xkv_ref[0], wk_ref[0], preferred_element_type=_F32)
    v_ref[0] = jnp.dot(xkv_ref[0], wv_ref[0], preferred_element_type=_F32)


def _proj(xs, Wqe_l, Wke_l, Wve_l):
    x_spec_q = pl.BlockSpec((1, BR, HID), lambda e, r: (1 - e, r, 0))
    x_spec = pl.BlockSpec((1, BR, HID), lambda e, r: (e, r, 0))
    w_spec = pl.BlockSpec((1, HID, HID), lambda e, r: (e, 0, 0))
    o_spec = pl.BlockSpec((1, BR, HID), lambda e, r: (e, r, 0))
    return pl.pallas_call(
        _proj_body,
        grid=(2, NBLK),
        in_specs=[x_spec_q, x_spec, w_spec, w_spec, w_spec],
        out_specs=[o_spec, o_spec, o_spec],
        out_shape=[jax.ShapeDtypeStruct((2, N, HID), _F32)] * 3,
    )(xs, xs, Wqe_l, Wke_l, Wve_l)


def _upd_body(n0_ref, d0_ref, n1_ref, d1_ref, xs_ref, wa_ref, beta_ref, o_ref):
    r = lax.broadcasted_iota(jnp.int32, (DH, HID), 0)
    c = lax.broadcasted_iota(jnp.int32, (DH, HID), 1)
    rep = (c // DH == r).astype(_F32)
    for t in range(2):
        num = (n0_ref, n1_ref)[t][0]
        den = (d0_ref, d1_ref)[t][0]
        deni = 1.0 / (den + 1e-9)
        deni_rep = jnp.dot(deni, rep, preferred_element_type=_F32)
        agg = num * deni_rep
        g = jax.nn.gelu(agg)
        a = jnp.dot(g, wa_ref[t], preferred_element_type=_F32)
        b = beta_ref[t][None, :]
        o_ref[t] = b * a + (1.0 - b) * xs_ref[t]


def _update(Pn, Pd, xs, Wa_l, beta2d):
    n0_spec = pl.BlockSpec((1, BR, HID), lambda r: (1, r, 0))
    d0_spec = pl.BlockSpec((1, BR, DH), lambda r: (1, r, 0))
    n1_spec = pl.BlockSpec((1, BR, HID), lambda r: (0, r, 0))
    d1_spec = pl.BlockSpec((1, BR, DH), lambda r: (0, r, 0))
    return pl.pallas_call(
        _upd_body,
        grid=(NBLK,),
        in_specs=[n0_spec, d0_spec, n1_spec, d1_spec,
                  pl.BlockSpec((2, BR, HID), lambda r: (0, r, 0)),
                  pl.BlockSpec((2, HID, HID), lambda r: (0, 0, 0)),
                  pl.BlockSpec((2, HID), lambda r: (0, 0))],
        out_specs=pl.BlockSpec((2, BR, HID), lambda r: (0, r, 0)),
        out_shape=jax.ShapeDtypeStruct((2, N, HID), _F32),
    )(Pn, Pd, Pn, Pd, xs, Wa_l, beta2d)


def _final_body(x_ref, w_ref, b_ref, o_ref):
    o_ref[...] = jnp.dot(x_ref[0], w_ref[...],
                         preferred_element_type=_F32) + b_ref[...]


def _final(xs, Wout, b_out2):
    return pl.pallas_call(
        _final_body,
        grid=(NBLK,),
        in_specs=[pl.BlockSpec((1, BR, HID), lambda r: (0, r, 0)),
                  pl.BlockSpec((HID, OUT), lambda r: (0, 0)),
                  pl.BlockSpec((1, OUT), lambda r: (0, 0))],
        out_specs=pl.BlockSpec((BR, OUT), lambda r: (r, 0)),
        out_shape=jax.ShapeDtypeStruct((N, OUT), _F32),
    )(xs, Wout, b_out2)


_sc_mesh = plsc.VectorSubcoreMesh(core_axis_name="c", subcore_axis_name="s",
                                  num_cores=NC, num_subcores=NS)
_sc_params = pltpu.CompilerParams(needs_layout_passes=False,
                                  use_tc_tiling_on_sc=False)


EPT = E // NS
CHM = 40
EPP = 10080
NCHT = EPP // CHM
UNR = 6
NIT = NCHT // UNR
ISZ = UNR * CHM


@functools.partial(
    pl.kernel,
    out_type=(jax.ShapeDtypeStruct((NC * N, HID), _F32),
              jax.ShapeDtypeStruct((NC * N, DH), _F32)),
    mesh=_sc_mesh,
    scratch_types=[
        pltpu.VMEM((2, ISZ), jnp.int32),
        pltpu.VMEM((2, ISZ), jnp.int32),
        pltpu.VMEM((3, CHM), jnp.int32),
        pltpu.VMEM((2, CHM), jnp.int32),
        pltpu.VMEM((2, CHM), jnp.int32),
        pltpu.VMEM((3, CHM), jnp.int32),
        pltpu.VMEM((2, CHM, HID), _F32),
        pltpu.VMEM((2, CHM, HID), _F32),
        pltpu.VMEM((3, CHM, HID), _F32),
        pltpu.VMEM((3, CHM, DH), _F32),
        pltpu.VMEM((256,), _F32),
        pltpu.VMEM_SHARED((N + 8, HID), _F32),
        pltpu.VMEM_SHARED((N + 8, DH), _F32),
        pltpu.SemaphoreType.DMA,
        pltpu.SemaphoreType.DMA,
        pltpu.SemaphoreType.DMA,
        pltpu.SemaphoreType.DMA,
        pltpu.SemaphoreType.DMA,
        pltpu.SemaphoreType.DMA,
        pltpu.SemaphoreType.DMA,
        pltpu.SemaphoreType.DMA,
        pltpu.SemaphoreType.DMA,
        pltpu.SemaphoreType.DMA,
        pltpu.SemaphoreType.DMA,
        pltpu.SemaphoreType.DMA,
        pltpu.SemaphoreType.DMA,
        pltpu.SemaphoreType.DMA,
        pltpu.SemaphoreType.DMA,
    ],
    compiler_params=_sc_params,
)
def _sc_edge(qtab, ktab, vtab, src, dst, zeros_n, zeros_d, outn, outd,
             rawS, rawD, sx, gq, gk, gv, qrows, krows, vrows, exden, stage,
             accn, accd, sq0, sq1, sk0, sk1, sv0, sv1, sv2,
             sa0, sa1, sa2, sb0, sb1, sb2, si0, si1):
    cid = lax.axis_index("c")
    sid = lax.axis_index("s")
    semq = (sq0, sq1)
    semk = (sk0, sk1)
    semv = (sv0, sv1, sv2)
    sema = (sa0, sa1, sa2)
    semb = (sb0, sb1, sb2)
    semi = (si0, si1)
    toff = cid * (NS * EPP) + sid * EPP

    r0 = sid * RPT
    pltpu.sync_copy(zeros_n.at[pl.ds(r0, RPT)], accn.at[pl.ds(r0, RPT)])
    pltpu.sync_copy(zeros_d.at[pl.ds(r0, RPT)], accd.at[pl.ds(r0, RPT)])
    plsc.subcore_barrier()

    lanes = lax.iota(jnp.int32, 16)
    g15 = lanes * 16 + 15
    rowoff = cid * N
    offs = list(range(0, CHM - 16, 16)) + [CHM - 16]

    def issue_idx_load(it, ib):
        base = toff + it * ISZ
        pltpu.async_copy(src.at[pl.ds(base, ISZ)], rawS.at[ib], semi[ib])
        pltpu.async_copy(dst.at[pl.ds(base, ISZ)], rawD.at[ib], semi[ib])

    def wait_idx(ib):
        pltpu.make_async_copy(src.at[pl.ds(toff, ISZ)], rawS.at[ib],
                              semi[ib]).wait()
        pltpu.make_async_copy(dst.at[pl.ds(toff, ISZ)], rawD.at[ib],
                              semi[ib]).wait()

    def stage_qk(loc, ib, b):
        for o in offs:
            dv_ = rawD[ib, pl.ds(loc * CHM + o, 16)]
            sv_ = rawS[ib, pl.ds(loc * CHM + o, 16)]
            gq[b, pl.ds(o, 16)] = jnp.minimum(dv_, N - 1) + rowoff
            gk[b, pl.ds(o, 16)] = jnp.minimum(sv_, N - 1) + rowoff
        pltpu.async_copy(qtab.at[gq.at[b]], qrows.at[b], semq[b])
        pltpu.async_copy(ktab.at[gk.at[b]], krows.at[b], semk[b])

    def stage_v(loc, ib, t):
        for o in offs:
            dv_ = rawD[ib, pl.ds(loc * CHM + o, 16)]
            sv_ = rawS[ib, pl.ds(loc * CHM + o, 16)]
            sx[t, pl.ds(o, 16)] = dv_
            gv[t, pl.ds(o, 16)] = jnp.minimum(sv_, N - 1) + rowoff
        pltpu.async_copy(vtab.at[gv.at[t]], vrows.at[t], semv[t])

    def wait_gathers_qk(b):
        pltpu.make_async_copy(qtab.at[gq.at[b]], qrows.at[b], semq[b]).wait()
        pltpu.make_async_copy(ktab.at[gk.at[b]], krows.at[b], semk[b]).wait()

    def wait_gather_v(t):
        pltpu.make_async_copy(vtab.at[gv.at[t]], vrows.at[t], semv[t]).wait()

    def issue_scatter(t):
        pltpu.async_copy(vrows.at[t], accn.at[sx.at[t]], sema[t], add=True)
        pltpu.async_copy(exden.at[t], accd.at[sx.at[t]], semb[t], add=True)

    def wait_scatter(t):
        pltpu.make_async_copy(vrows.at[t], accn.at[sx.at[t]], sema[t]).wait()
        pltpu.make_async_copy(exden.at[t], accd.at[sx.at[t]], semb[t]).wait()

    def compute(b, t):
        qr, kr, vr, er = qrows.at[b], krows.at[b], vrows.at[t], exden.at[t]

        def edge_body(i, cc):
            for h in range(H):
                qv = qr[i, pl.ds(h * DH, DH)]
                kv = kr[i, pl.ds(h * DH, DH)]
                stage[pl.ds(h * DH, DH)] = plsc.cumsum(qv * kv)
            tot = plsc.load_gather(stage, [g15])
            sc = jnp.clip(tot, -60.0, 60.0)
            ex = jnp.where(lanes < H, jnp.exp(sc), 0.0)
            er[i, pl.ds(0, DH)] = ex
            for h in range(H):
                vr[i, pl.ds(h * DH, DH)] = vr[i, pl.ds(h * DH, DH)] * ex[h]
            return cc

        lax.fori_loop(0, CHM, edge_body, 0)

    issue_idx_load(0, 0)
    wait_idx(0)
    issue_idx_load(1, 1)
    stage_qk(0, 0, 0)
    stage_v(0, 0, 0)
    stage_qk(1, 0, 1)
    stage_v(1, 0, 1)

    def iter_pair_body(itp, carry):
        for ip in range(2):
            it = 2 * itp + ip
            ib, nxt = ip, 1 - ip

            def _issue_next_idx():
                issue_idx_load(it + 1, nxt)
            if ip == 0:
                @pl.when(itp >= 1)
                def _():
                    _issue_next_idx()
            else:
                @pl.when(itp < NIT // 2 - 1)
                def _():
                    _issue_next_idx()

            for u in range(UNR):
                b = u % 2
                t = u % 3
                tn = (u + 2) % 3
                wait_gathers_qk(b)
                wait_gather_v(t)
                compute(b, t)
                if ip == 0 and u == 0:
                    @pl.when(itp >= 1)
                    def _():
                        wait_scatter(tn)
                else:
                    wait_scatter(tn)
                if u < UNR - 2:
                    stage_qk(u + 2, ib, b)
                    stage_v(u + 2, ib, tn)
                elif u == UNR - 2:
                    def _stage_from_next(loc=0, b_=b, tn_=tn):
                        wait_idx(nxt)
                        stage_qk(loc, nxt, b_)
                        stage_v(loc, nxt, tn_)
                    if ip == 0:
                        _stage_from_next()
                    else:
                        @pl.when(itp < NIT // 2 - 1)
                        def _():
                            _stage_from_next()
                else:
                    def _stage_from_next2(loc=1, b_=b, tn_=tn):
                        stage_qk(loc, nxt, b_)
                        stage_v(loc, nxt, tn_)
                    if ip == 0:
                        _stage_from_next2()
                    else:
                        @pl.when(itp < NIT // 2 - 1)
                        def _():
                            _stage_from_next2()
                issue_scatter(t)
        return carry

    lax.fori_loop(0, NIT // 2, iter_pair_body, 0)
    wait_scatter((NCHT - 1) % 3)

    plsc.subcore_barrier()
    pltpu.sync_copy(accn.at[pl.ds(r0, RPT)],
                    outn.at[pl.ds(cid * N + r0, RPT)])
    pltpu.sync_copy(accd.at[pl.ds(r0, RPT)],
                    outd.at[pl.ds(cid * N + r0, RPT)])


def kernel(x_user, x_item, edge_index_u2i, edge_index_i2u, Win, b_in,
           Wk, Wq, Wv, Wa, a_rel, m_rel, mu, skip, Wout, b_out):
    X = jnp.stack([x_user, x_item])
    xs = _input_proj(X, Win, b_in.reshape(2, 1, HID))

    eye = jnp.eye(H, dtype=_F32)
    def _bd(a):
        a6 = a[:, :, :, :, None, :] * eye[None, None, :, None, :, None]
        return a6.reshape(NL, 2, HID, HID)

    Wqe, Wke, Wve = _eff_weights(Wq, Wk, Wv, _bd(a_rel), _bd(m_rel),
                                 mu.reshape(NL, 2, 1, H))

    def _pad_edges(ei):
        sp = jnp.pad(ei[0].reshape(NS, EPT), ((0, 0), (0, EPP - EPT)))
        dp = jnp.pad(ei[1].reshape(NS, EPT), ((0, 0), (0, EPP - EPT)),
                     constant_values=N)
        return sp.reshape(-1), dp.reshape(-1)

    s0, d0 = _pad_edges(edge_index_u2i)
    s1, d1 = _pad_edges(edge_index_i2u)
    srccat = jnp.concatenate([s0, s1])
    dstcat = jnp.concatenate([d0, d1])
    zeros_n = jnp.zeros((N, HID), _F32)
    zeros_d = jnp.zeros((N, DH), _F32)
    beta = jax.nn.sigmoid(skip)

    for l in range(NL):
        Q, K, V = _proj(xs, Wqe[l], Wke[l], Wve[l])
        Pn, Pd = _sc_edge(Q.reshape(NC * N, HID), K.reshape(NC * N, HID),
                          V.reshape(NC * N, HID), srccat, dstcat,
                          zeros_n, zeros_d)
        beta2d = jnp.broadcast_to(beta[l][:, None], (2, HID))
        xs = _update(Pn.reshape(NC, N, HID), Pd.reshape(NC, N, DH),
                     xs, Wa[l], beta2d)

    return _final(xs, Wout, b_out.reshape(1, OUT))
